# baseline (speedup 1.0000x reference)
import functools

import jax
import jax.numpy as jnp
from jax import lax
from jax.experimental import pallas as pl
from jax.experimental.pallas import tpu as pltpu
from jax.experimental.pallas import tpu_sc as plsc

N = 10000
E = 320000
D = 128

NC = 2
NS = 16
NW = NC * NS
EPW = E // NW
CHUNK = 80
NCHUNK = EPW // CHUNK
NBUF = 4
NITER = 31
GROUPS = CHUNK // 16
NP = 10240
RPT = NP // NS
LANES = 16
LAST = NCHUNK - 1


def _lane_splat(vec, i):
    idx = jnp.full((LANES, 1), i, jnp.int32)
    dnums = lax.GatherDimensionNumbers(
        offset_dims=(), collapsed_slice_dims=(0,), start_index_map=(0,))
    return lax.gather(vec, idx, dnums, (1,),
                      mode=lax.GatherScatterMode.PROMISE_IN_BOUNDS)


def _sc_partials(adji, val, embeds):
    mesh = plsc.VectorSubcoreMesh(core_axis_name="c", subcore_axis_name="s")

    @functools.partial(
        pl.kernel,
        mesh=mesh,
        out_type=jax.ShapeDtypeStruct((NC, NP, D), jnp.float32),
        scratch_types=(
            [pltpu.VMEM_SHARED((NP, D), jnp.float32)]
            + [pltpu.VMEM((CHUNK, D), jnp.float32) for _ in range(NBUF)]
            + [pltpu.VMEM((2, CHUNK), jnp.int32) for _ in range(NBUF)]
            + [pltpu.VMEM((CHUNK,), jnp.float32) for _ in range(NBUF)]
            + [pltpu.SemaphoreType.DMA for _ in range(4 * NBUF)]
        ),
    )
    def k(adji_hbm, val_hbm, emb_hbm, out_hbm, acc, *bufs_sems):
        bufs = bufs_sems[:NBUF]
        mb = bufs_sems[NBUF:2 * NBUF]
        mbv = bufs_sems[2 * NBUF:3 * NBUF]
        gsem = bufs_sems[3 * NBUF:4 * NBUF]
        ssem = bufs_sems[4 * NBUF:5 * NBUF]
        msem = bufs_sems[5 * NBUF:6 * NBUF]
        vsem = bufs_sems[6 * NBUF:]
        c = lax.axis_index("c")
        s = lax.axis_index("s")
        wid = c * NS + s

        zero = jnp.zeros((LANES,), jnp.float32)
        for i in range(CHUNK):
            for j in range(D // LANES):
                bufs[0][i, pl.ds(j * LANES, LANES)] = zero
        for t in range(RPT // CHUNK):
            pltpu.sync_copy(bufs[0],
                            acc.at[pl.ds(s * RPT + t * CHUNK, CHUNK)])
        plsc.subcore_barrier()

        ebase = wid * EPW

        def start_meta(kk, b):
            pltpu.async_copy(
                adji_hbm.at[pl.ds(ebase + kk * CHUNK, CHUNK)],
                mb[b].at[0], msem[b])
            pltpu.async_copy(
                adji_hbm.at[pl.ds(E + ebase + kk * CHUNK, CHUNK)],
                mb[b].at[1], msem[b])

        def wait_meta(kk, b):
            pltpu.make_async_copy(
                adji_hbm.at[pl.ds(ebase + kk * CHUNK, CHUNK)],
                mb[b].at[0], msem[b]).wait()
            pltpu.make_async_copy(
                adji_hbm.at[pl.ds(E + ebase + kk * CHUNK, CHUNK)],
                mb[b].at[1], msem[b]).wait()

        def start_val(kk, b):
            pltpu.async_copy(val_hbm.at[pl.ds(ebase + kk * CHUNK, CHUNK)],
                             mbv[b], vsem[b])

        def wait_val(kk, b):
            pltpu.make_async_copy(
                val_hbm.at[pl.ds(ebase + kk * CHUNK, CHUNK)],
                mbv[b], vsem[b]).wait()

        def start_gather(kk, b):
            pltpu.async_copy(emb_hbm.at[mb[b].at[1]], bufs[b], gsem[b])

        def wait_gather(kk, b):
            pltpu.make_async_copy(
                emb_hbm.at[mb[b].at[1]], bufs[b], gsem[b]).wait()

        def start_scat(kk, b):
            pltpu.async_copy(bufs[b], acc.at[mb[b].at[0]], ssem[b],
                             add=True)

        def wait_scat(kk, b):
            pltpu.make_async_copy(
                bufs[b], acc.at[mb[b].at[0]], ssem[b]).wait()

        def scale(kk, b):
            def gbody(g, carry):
                val16 = mbv[b][pl.ds(g * LANES, LANES)]
                for i in range(LANES):
                    e = g * LANES + i
                    vsplat = _lane_splat(val16, i)
                    for j in range(D // LANES):
                        sl = pl.ds(j * LANES, LANES)
                        bufs[b][e, sl] = bufs[b][e, sl] * vsplat
                return carry
            lax.fori_loop(0, GROUPS, gbody, 0)

        for j in range(NBUF):
            start_meta(j, j)
            start_val(j, j)
        for j in range(2):
            wait_meta(j, j)
            start_gather(j, j)

        def chunk_body(m, carry):
            for j in range(NBUF):
                kk = m * NBUF + j
                nj = (j + 2) % NBUF
                pj = (j - 1) % NBUF

                wait_gather(kk, j)
                wait_val(kk, j)
                scale(kk, j)

                @pl.when(kk + NBUF <= LAST)
                def _():
                    start_val(kk + NBUF, j)

                start_scat(kk, j)

                if j == 0:
                    @pl.when(m > 0)
                    def _():
                        wait_scat(kk - 1, pj)
                        start_meta(kk + NBUF - 1, pj)
                else:
                    wait_scat(kk - 1, pj)

                    @pl.when(kk + NBUF - 1 <= LAST)
                    def _():
                        start_meta(kk + NBUF - 1, pj)

                @pl.when(kk + 2 <= LAST)
                def _():
                    wait_meta(kk + 2, nj)
                    start_gather(kk + 2, nj)
            return carry

        lax.fori_loop(0, NITER, chunk_body, 0)

        wait_scat(LAST - 1, (LAST - 1) % NBUF)
        wait_gather(LAST, 0)
        wait_val(LAST, 0)
        scale(LAST, 0)
        start_scat(LAST, 0)
        wait_scat(LAST, 0)

        plsc.subcore_barrier()
        pltpu.sync_copy(acc.at[pl.ds(s * RPT, RPT)],
                        out_hbm.at[c, pl.ds(s * RPT, RPT)])

    return k(adji, val, embeds)


def _combine(partials):
    def body(p_ref, o_ref):
        o_ref[...] = p_ref[0] + p_ref[1]

    rblk = 2000
    return pl.pallas_call(
        body,
        out_shape=jax.ShapeDtypeStruct((N, D), jnp.float32),
        grid=(N // rblk,),
        in_specs=[pl.BlockSpec((NC, rblk, D), lambda i: (0, i, 0))],
        out_specs=pl.BlockSpec((rblk, D), lambda i: (i, 0)),
    )(partials)


def kernel(adj_indices, adj_values, embeds):
    partials = _sc_partials(adj_indices.reshape(2 * E), adj_values, embeds)
    return _combine(partials)


# speedup vs baseline: 1.0399x; 1.0399x over previous
import functools

import jax
import jax.numpy as jnp
from jax import lax
from jax.experimental import pallas as pl
from jax.experimental.pallas import tpu as pltpu
from jax.experimental.pallas import tpu_sc as plsc

N = 10000
E = 320000
D = 128

NC = 2
NS = 16
NW = NC * NS
EPW = E // NW
CHUNK = 80
NCHUNK = EPW // CHUNK
NBUF = 4
NITER = 31
GROUPS = CHUNK // 16
NP = 10240
RPT = NP // NS
LANES = 16


def _lane_splat(vec, i):
    idx = jnp.full((LANES, 1), i, jnp.int32)
    dnums = lax.GatherDimensionNumbers(
        offset_dims=(), collapsed_slice_dims=(0,), start_index_map=(0,))
    return lax.gather(vec, idx, dnums, (1,),
                      mode=lax.GatherScatterMode.PROMISE_IN_BOUNDS)


def _sc_partials(adji_flat, val, embeds):
    mesh = plsc.VectorSubcoreMesh(core_axis_name="c", subcore_axis_name="s")

    @functools.partial(
        pl.kernel,
        mesh=mesh,
        out_type=jax.ShapeDtypeStruct((NC, NP, D), jnp.float32),
        scratch_types=(
            [pltpu.VMEM_SHARED((NP, D), jnp.float32)]
            + [pltpu.VMEM((CHUNK, D), jnp.float32) for _ in range(NBUF)]
            + [pltpu.VMEM((CHUNK,), jnp.int32) for _ in range(NBUF)]
            + [pltpu.VMEM((CHUNK,), jnp.float32) for _ in range(NBUF)]
            + [pltpu.VMEM((CHUNK,), jnp.int32) for _ in range(NBUF)]
            + [pltpu.SemaphoreType.DMA for _ in range(5 * NBUF)]
        ),
    )
    def k(adji_hbm, val_hbm, emb_hbm, out_hbm, acc, *bufs_sems):
        bufs = bufs_sems[:NBUF]
        mbc = bufs_sems[NBUF:2 * NBUF]
        mbv = bufs_sems[2 * NBUF:3 * NBUF]
        mbr = bufs_sems[3 * NBUF:4 * NBUF]
        gsem = bufs_sems[4 * NBUF:5 * NBUF]
        ssem = bufs_sems[5 * NBUF:6 * NBUF]
        csem = bufs_sems[6 * NBUF:7 * NBUF]
        vsem = bufs_sems[7 * NBUF:8 * NBUF]
        rsem = bufs_sems[8 * NBUF:]
        c = lax.axis_index("c")
        s = lax.axis_index("s")
        wid = c * NS + s

        zero = jnp.zeros((LANES,), jnp.float32)
        for i in range(CHUNK):
            for j in range(D // LANES):
                bufs[0][i, pl.ds(j * LANES, LANES)] = zero
        for t in range(RPT // CHUNK):
            pltpu.sync_copy(bufs[0],
                            acc.at[pl.ds(s * RPT + t * CHUNK, CHUNK)])
        plsc.subcore_barrier()

        ebase = wid * EPW

        def start_col(kk, b):
            pltpu.async_copy(
                adji_hbm.at[pl.ds(E + ebase + kk * CHUNK, CHUNK)],
                mbc[b], csem[b])

        def wait_col(kk, b):
            pltpu.make_async_copy(
                adji_hbm.at[pl.ds(E + ebase + kk * CHUNK, CHUNK)],
                mbc[b], csem[b]).wait()

        def start_val(kk, b):
            pltpu.async_copy(val_hbm.at[pl.ds(ebase + kk * CHUNK, CHUNK)],
                             mbv[b], vsem[b])

        def wait_val(kk, b):
            pltpu.make_async_copy(
                val_hbm.at[pl.ds(ebase + kk * CHUNK, CHUNK)],
                mbv[b], vsem[b]).wait()

        def start_row(kk, b):
            pltpu.async_copy(adji_hbm.at[pl.ds(ebase + kk * CHUNK, CHUNK)],
                             mbr[b], rsem[b])

        def wait_row(kk, b):
            pltpu.make_async_copy(
                adji_hbm.at[pl.ds(ebase + kk * CHUNK, CHUNK)],
                mbr[b], rsem[b]).wait()

        def start_gather(kk, b):
            pltpu.async_copy(emb_hbm.at[mbc[b]], bufs[b], gsem[b])

        def wait_gather(kk, b):
            pltpu.make_async_copy(
                emb_hbm.at[mbc[b]], bufs[b], gsem[b]).wait()

        def start_scat(kk, b):
            pltpu.async_copy(bufs[b], acc.at[mbr[b]], ssem[b], add=True)

        def wait_scat(kk, b):
            pltpu.make_async_copy(
                bufs[b], acc.at[mbr[b]], ssem[b]).wait()

        def scale(kk, b):
            def gbody(g, carry):
                val16 = mbv[b][pl.ds(g * LANES, LANES)]
                for i in range(LANES):
                    e = g * LANES + i
                    vsplat = _lane_splat(val16, i)
                    for j in range(D // LANES):
                        sl = pl.ds(j * LANES, LANES)
                        bufs[b][e, sl] = bufs[b][e, sl] * vsplat
                return carry
            lax.fori_loop(0, GROUPS, gbody, 0)

        for j in range(NBUF):
            start_col(j, j)
            start_val(j, j)
            start_row(j, j)
        for j in range(NBUF):
            wait_col(j, j)
            start_gather(j, j)

        def chunk_body(m, carry):
            for j in range(NBUF):
                kk = m * NBUF + j
                wait_gather(kk, j)

                @pl.when(kk + NBUF <= NCHUNK - 1)
                def _():
                    start_col(kk + NBUF, j)

                wait_val(kk, j)
                scale(kk, j)

                @pl.when(kk + NBUF <= NCHUNK - 1)
                def _():
                    start_val(kk + NBUF, j)

                wait_row(kk, j)
                start_scat(kk, j)

                pj = (j - 1) % NBUF
                if j == 0:
                    @pl.when(m > 0)
                    def _():
                        wait_scat(kk - 1, pj)
                        start_row(kk + NBUF - 1, pj)
                        wait_col(kk + NBUF - 1, pj)
                        start_gather(kk + NBUF - 1, pj)
                else:
                    wait_scat(kk - 1, pj)

                    @pl.when(kk + NBUF - 1 <= NCHUNK - 1)
                    def _():
                        start_row(kk + NBUF - 1, pj)
                        wait_col(kk + NBUF - 1, pj)
                        start_gather(kk + NBUF - 1, pj)
            return carry

        lax.fori_loop(0, NITER, chunk_body, 0)

        last = NCHUNK - 1
        wait_scat(last - 1, (last - 1) % NBUF)
        wait_gather(last, 0)
        wait_val(last, 0)
        scale(last, 0)
        wait_row(last, 0)
        start_scat(last, 0)
        wait_scat(last, 0)

        plsc.subcore_barrier()
        pltpu.sync_copy(acc.at[pl.ds(s * RPT, RPT)],
                        out_hbm.at[c, pl.ds(s * RPT, RPT)])

    return k(adji_flat, val, embeds)


def _combine(partials):
    def body(p_ref, o_ref):
        o_ref[...] = p_ref[0] + p_ref[1]

    rblk = 2000
    return pl.pallas_call(
        body,
        out_shape=jax.ShapeDtypeStruct((N, D), jnp.float32),
        grid=(N // rblk,),
        in_specs=[pl.BlockSpec((NC, rblk, D), lambda i: (0, i, 0))],
        out_specs=pl.BlockSpec((rblk, D), lambda i: (i, 0)),
    )(partials)


def kernel(adj_indices, adj_values, embeds):
    partials = _sc_partials(adj_indices.reshape(2 * E), adj_values, embeds)
    return _combine(partials)
